# Initial kernel scaffold; baseline (speedup 1.0000x reference)
import jax
import jax.numpy as jnp
from jax.experimental import pallas as pl


def kernel(positions, clip_scales, hash_table, W0, W1, W2, W3):
    raise NotImplementedError("write your pallas kernel here")


# speedup vs baseline: 2.0747x; 2.0747x over previous
import numpy as np
import jax
import jax.numpy as jnp
from jax import lax
from jax.experimental import pallas as pl
from jax.experimental.pallas import tpu as pltpu
from jax.experimental.pallas import tpu_sc as plsc

_N_LEVELS = 12
_TSIZE = 2 ** 19
_BASE, _END = 16, 128
_GROWTH = float(np.exp((np.log(_END) - np.log(_BASE)) / (_N_LEVELS - 1)))
_RES = [int(np.floor(_BASE * _GROWTH ** l)) for l in range(_N_LEVELS)]
_P1 = int(np.uint32(2654435761).view(np.int32))
_P2 = int(np.uint32(805459861).view(np.int32))
_MASK = _TSIZE - 1

_NC, _NS = 2, 16
_NW = _NC * _NS
_C = 128
_ROWS = _N_LEVELS * 8
_FIRE = 8


def _sc_encode_body(nsub, pos_ref, tab_ref, out_ref,
                    posv, idxv, wv, valv, featv, gsem):
    cid = lax.axis_index("c")
    sid = lax.axis_index("s")
    wid = sid * _NC + cid
    pts_per_w = nsub * _C

    def sub_body(sub, carry):
        base = wid * pts_per_w + sub * _C
        pltpu.sync_copy(pos_ref.at[:, pl.ds(base, _C)], posv)

        def group_body(g, carry2):
            off = g * 16
            ux = posv[0, pl.ds(off, 16)]
            uy = posv[1, pl.ds(off, 16)]
            uz = posv[2, pl.ds(off, 16)]
            for l in range(_N_LEVELS):
                r = jnp.float32(_RES[l])
                x = ux * r
                y = uy * r
                z = uz * r
                xi = x.astype(jnp.int32)
                yi = y.astype(jnp.int32)
                zi = z.astype(jnp.int32)
                fx = x - xi.astype(jnp.float32)
                fy = y - yi.astype(jnp.float32)
                fz = z - zi.astype(jnp.float32)
                hy0 = yi * jnp.int32(_P1)
                hy1 = hy0 + jnp.int32(_P1)
                hz0 = zi * jnp.int32(_P2)
                hz1 = hz0 + jnp.int32(_P2)
                xy = (xi ^ hy0, (xi + 1) ^ hy0, xi ^ hy1, (xi + 1) ^ hy1)
                gx = (1.0 - fx, fx)
                gy = (1.0 - fy, fy)
                gz = (1.0 - fz, fz)
                gxy = (gx[0] * gy[0], gx[1] * gy[0], gx[0] * gy[1], gx[1] * gy[1])
                lbase = jnp.int32(l * _TSIZE)
                c = 0
                for dz in (0, 1):
                    hz = hz0 if dz == 0 else hz1
                    for dy in (0, 1):
                        for dx in (0, 1):
                            h = ((xy[dy * 2 + dx] ^ hz) & jnp.int32(_MASK)) + lbase
                            idxv[l * 8 + c, pl.ds(off, 16)] = h
                            wv[l * 8 + c, pl.ds(off, 16)] = gxy[dy * 2 + dx] * gz[dz]
                            c += 1
            return carry2

        lax.fori_loop(0, _C // 16, group_body, 0)

        def round_body(rr, carry2):
            def fire(i, c3):
                j = rr * _FIRE + i
                pltpu.make_async_copy(tab_ref.at[idxv.at[j]], valv.at[j], gsem).start()
                return c3

            lax.fori_loop(0, _FIRE, fire, 0)

            def drain(i, c3):
                j = rr * _FIRE + i
                pltpu.make_async_copy(tab_ref.at[idxv.at[j]], valv.at[j], gsem).wait()
                return c3

            lax.fori_loop(0, _FIRE, drain, 0)
            return carry2

        lax.fori_loop(0, _ROWS // _FIRE, round_body, 0)

        def acc_body(g, carry2):
            off = g * 16
            for l in range(_N_LEVELS):
                acc = valv[l * 8, pl.ds(off, 16)] * wv[l * 8, pl.ds(off, 16)]
                for c in range(1, 8):
                    acc = acc + valv[l * 8 + c, pl.ds(off, 16)] * wv[l * 8 + c, pl.ds(off, 16)]
                featv[l, pl.ds(off, 16)] = acc
            return carry2

        lax.fori_loop(0, _C // 16, acc_body, 0)

        pltpu.sync_copy(featv, out_ref.at[:, pl.ds(base, _C)])
        return carry

    lax.fori_loop(0, nsub, sub_body, 0)


def _sc_encode(pos_T, table_flat):
    n = pos_T.shape[1]
    nsub = n // (_NW * _C)
    mesh = plsc.VectorSubcoreMesh(core_axis_name="c", subcore_axis_name="s",
                                  num_cores=_NC, num_subcores=_NS)
    body = lambda *refs: _sc_encode_body(nsub, *refs)
    return pl.kernel(
        body,
        out_type=jax.ShapeDtypeStruct((_N_LEVELS, n), jnp.float32),
        mesh=mesh,
        scratch_types=[
            pltpu.VMEM((3, _C), jnp.float32),
            pltpu.VMEM((_ROWS, _C), jnp.int32),
            pltpu.VMEM((_ROWS, _C), jnp.float32),
            pltpu.VMEM((_ROWS, _C), jnp.float32),
            pltpu.VMEM((_N_LEVELS, _C), jnp.float32),
            pltpu.SemaphoreType.DMA,
        ],
    )(pos_T, table_flat)


def _prep_body(pos_ref, out_ref):
    p = pos_ref[...]
    s = jnp.sum(p * p, axis=0, keepdims=True)
    rs = lax.rsqrt(jnp.maximum(s, 1.0))
    scale = jnp.where(s > 1.0, (2.0 - rs) * rs, 1.0)
    out_ref[...] = (p * scale + 2.0) * 0.25


def _prep(pos_T):
    n = pos_T.shape[1]
    nb = 8192
    return pl.pallas_call(
        _prep_body,
        grid=(n // nb,),
        in_specs=[pl.BlockSpec((3, nb), lambda i: (0, i))],
        out_specs=pl.BlockSpec((3, nb), lambda i: (0, i)),
        out_shape=jax.ShapeDtypeStruct((3, n), jnp.float32),
    )(pos_T)


def _mlp_body(feat_ref, cs_ref, w0_ref, w1_ref, w2_ref, w3_ref, clip_ref, hg_ref):
    ft = feat_ref[...]
    cs = cs_ref[...]
    w0 = w0_ref[...]
    h = lax.dot_general(ft, w0[0:12, :], (((0,), (0,)), ((), ())),
                        preferred_element_type=jnp.float32)
    h = jnp.maximum(h + cs * w0[12:13, :], 0.0)
    h = jnp.maximum(jnp.dot(h, w1_ref[...], preferred_element_type=jnp.float32), 0.0)
    h = jnp.maximum(jnp.dot(h, w2_ref[...], preferred_element_type=jnp.float32), 0.0)
    o = jnp.dot(h, w3_ref[...], preferred_element_type=jnp.float32)
    ssq = jnp.sum(o * o, axis=1, keepdims=True)
    clip_ref[...] = o * lax.rsqrt(ssq)
    hg_ref[...] = ft.T


def _mlp(feats_T, clip_scales, w0t, w1t, w2t, w3t):
    n = feats_T.shape[1]
    nb = 2048
    return pl.pallas_call(
        _mlp_body,
        grid=(n // nb,),
        in_specs=[
            pl.BlockSpec((_N_LEVELS, nb), lambda i: (0, i)),
            pl.BlockSpec((nb, 1), lambda i: (i, 0)),
            pl.BlockSpec((13, 256), lambda i: (0, 0)),
            pl.BlockSpec((256, 256), lambda i: (0, 0)),
            pl.BlockSpec((256, 256), lambda i: (0, 0)),
            pl.BlockSpec((256, 512), lambda i: (0, 0)),
        ],
        out_specs=[
            pl.BlockSpec((nb, 512), lambda i: (i, 0)),
            pl.BlockSpec((nb, _N_LEVELS), lambda i: (i, 0)),
        ],
        out_shape=[
            jax.ShapeDtypeStruct((n, 512), jnp.float32),
            jax.ShapeDtypeStruct((n, _N_LEVELS), jnp.float32),
        ],
    )(feats_T, clip_scales, w0t, w1t, w2t, w3t)


def kernel(positions, clip_scales, hash_table, W0, W1, W2, W3):
    pos_T = positions.T
    table_flat = hash_table.reshape(-1)
    u_T = _prep(pos_T)
    feats_T = _sc_encode(u_T, table_flat)
    clip, hashgrid = _mlp(feats_T, clip_scales.reshape(-1, 1),
                          W0.T, W1.T, W2.T, W3.T)
    return hashgrid, clip


# speedup vs baseline: 2.9026x; 1.3990x over previous
import numpy as np
import jax
import jax.numpy as jnp
from jax import lax
from jax.experimental import pallas as pl
from jax.experimental.pallas import tpu as pltpu
from jax.experimental.pallas import tpu_sc as plsc

_N_LEVELS = 12
_TSIZE = 2 ** 19
_BASE, _END = 16, 128
_GROWTH = float(np.exp((np.log(_END) - np.log(_BASE)) / (_N_LEVELS - 1)))
_RES = [int(np.floor(_BASE * _GROWTH ** l)) for l in range(_N_LEVELS)]
_P1 = int(np.uint32(2654435761).view(np.int32))
_P2 = int(np.uint32(805459861).view(np.int32))
_MASK = _TSIZE - 1

_NC, _NS = 2, 16
_NW = _NC * _NS
_C = 128
_ROWS = _N_LEVELS * 8
_FIRE = 8


def _sc_encode_body(nsub, pos_ref, tab_ref, out_ref,
                    posv, idxA, wA, valA, idxB, wB, valB, featv, semA, semB):
    cid = lax.axis_index("c")
    sid = lax.axis_index("s")
    wid = sid * _NC + cid
    pts_per_w = nsub * _C
    base0 = wid * pts_per_w
    pltpu.sync_copy(pos_ref.at[:, pl.ds(base0, pts_per_w)], posv)

    def compute(sub, idxv, wv):
        def group_body(g, carry2):
            off = sub * _C + g * 16
            soff = g * 16
            ux = posv[0, pl.ds(off, 16)]
            uy = posv[1, pl.ds(off, 16)]
            uz = posv[2, pl.ds(off, 16)]
            for l in range(_N_LEVELS):
                r = jnp.float32(_RES[l])
                x = ux * r
                y = uy * r
                z = uz * r
                xi = x.astype(jnp.int32)
                yi = y.astype(jnp.int32)
                zi = z.astype(jnp.int32)
                fx = x - xi.astype(jnp.float32)
                fy = y - yi.astype(jnp.float32)
                fz = z - zi.astype(jnp.float32)
                hy0 = yi * jnp.int32(_P1)
                hy1 = hy0 + jnp.int32(_P1)
                hz0 = zi * jnp.int32(_P2)
                hz1 = hz0 + jnp.int32(_P2)
                xy = (xi ^ hy0, (xi + 1) ^ hy0, xi ^ hy1, (xi + 1) ^ hy1)
                gx = (1.0 - fx, fx)
                gy = (1.0 - fy, fy)
                gz = (1.0 - fz, fz)
                gxy = (gx[0] * gy[0], gx[1] * gy[0], gx[0] * gy[1], gx[1] * gy[1])
                lbase = jnp.int32(l * _TSIZE)
                c = 0
                for dz in (0, 1):
                    hz = hz0 if dz == 0 else hz1
                    for dy in (0, 1):
                        for dx in (0, 1):
                            h = ((xy[dy * 2 + dx] ^ hz) & jnp.int32(_MASK)) + lbase
                            idxv[l * 8 + c, pl.ds(soff, 16)] = h
                            wv[l * 8 + c, pl.ds(soff, 16)] = gxy[dy * 2 + dx] * gz[dz]
                            c += 1
            return carry2

        lax.fori_loop(0, _C // 16, group_body, 0)

    def fire(idxv, valv, sem):
        def f(j, c3):
            pltpu.make_async_copy(tab_ref.at[idxv.at[j]], valv.at[j], sem).start()
            return c3

        lax.fori_loop(0, _ROWS, f, 0)

    def drain_acc(sub, idxv, wv, valv, sem):
        def dr(j, c3):
            pltpu.make_async_copy(tab_ref.at[idxv.at[j]], valv.at[j], sem).wait()
            return c3

        lax.fori_loop(0, _ROWS, dr, 0)

        def acc_body(g, carry2):
            off = g * 16
            for l in range(_N_LEVELS):
                acc = valv[l * 8, pl.ds(off, 16)] * wv[l * 8, pl.ds(off, 16)]
                for c in range(1, 8):
                    acc = acc + valv[l * 8 + c, pl.ds(off, 16)] * wv[l * 8 + c, pl.ds(off, 16)]
                featv[l, pl.ds(off, 16)] = acc
            return carry2

        lax.fori_loop(0, _C // 16, acc_body, 0)
        pltpu.sync_copy(featv, out_ref.at[:, pl.ds(base0 + sub * _C, _C)])

    compute(0, idxA, wA)
    fire(idxA, valA, semA)

    def pair(p, carry):
        s0 = 2 * p
        compute(s0 + 1, idxB, wB)
        fire(idxB, valB, semB)
        drain_acc(s0, idxA, wA, valA, semA)
        compute(s0 + 2, idxA, wA)
        fire(idxA, valA, semA)
        drain_acc(s0 + 1, idxB, wB, valB, semB)
        return carry

    lax.fori_loop(0, nsub // 2 - 1, pair, 0)
    compute(nsub - 1, idxB, wB)
    fire(idxB, valB, semB)
    drain_acc(nsub - 2, idxA, wA, valA, semA)
    drain_acc(nsub - 1, idxB, wB, valB, semB)


def _sc_encode(pos_T, table_flat):
    n = pos_T.shape[1]
    nsub = n // (_NW * _C)
    mesh = plsc.VectorSubcoreMesh(core_axis_name="c", subcore_axis_name="s",
                                  num_cores=_NC, num_subcores=_NS)
    body = lambda *refs: _sc_encode_body(nsub, *refs)
    return pl.kernel(
        body,
        out_type=jax.ShapeDtypeStruct((_N_LEVELS, n), jnp.float32),
        mesh=mesh,
        scratch_types=[
            pltpu.VMEM((3, nsub * _C), jnp.float32),
            pltpu.VMEM((_ROWS, _C), jnp.int32),
            pltpu.VMEM((_ROWS, _C), jnp.float32),
            pltpu.VMEM((_ROWS, _C), jnp.float32),
            pltpu.VMEM((_ROWS, _C), jnp.int32),
            pltpu.VMEM((_ROWS, _C), jnp.float32),
            pltpu.VMEM((_ROWS, _C), jnp.float32),
            pltpu.VMEM((_N_LEVELS, _C), jnp.float32),
            pltpu.SemaphoreType.DMA,
            pltpu.SemaphoreType.DMA,
        ],
    )(pos_T, table_flat)


def _prep_body(pos_ref, out_ref):
    p = pos_ref[...]
    s = jnp.sum(p * p, axis=0, keepdims=True)
    rs = lax.rsqrt(jnp.maximum(s, 1.0))
    scale = jnp.where(s > 1.0, (2.0 - rs) * rs, 1.0)
    out_ref[...] = (p * scale + 2.0) * 0.25


def _prep(pos_T):
    n = pos_T.shape[1]
    nb = 8192
    return pl.pallas_call(
        _prep_body,
        grid=(n // nb,),
        in_specs=[pl.BlockSpec((3, nb), lambda i: (0, i))],
        out_specs=pl.BlockSpec((3, nb), lambda i: (0, i)),
        out_shape=jax.ShapeDtypeStruct((3, n), jnp.float32),
    )(pos_T)


def _mlp_body(feat_ref, cs_ref, w0_ref, w1_ref, w2_ref, w3_ref, clip_ref, hg_ref):
    ft = feat_ref[...]
    cs = cs_ref[...]
    w0 = w0_ref[...]
    h = lax.dot_general(ft, w0[0:12, :], (((0,), (0,)), ((), ())),
                        preferred_element_type=jnp.float32)
    h = jnp.maximum(h + cs * w0[12:13, :], 0.0)
    h = jnp.maximum(jnp.dot(h, w1_ref[...], preferred_element_type=jnp.float32), 0.0)
    h = jnp.maximum(jnp.dot(h, w2_ref[...], preferred_element_type=jnp.float32), 0.0)
    o = jnp.dot(h, w3_ref[...], preferred_element_type=jnp.float32)
    ssq = jnp.sum(o * o, axis=1, keepdims=True)
    clip_ref[...] = o * lax.rsqrt(ssq)
    hg_ref[...] = ft.T


def _mlp(feats_T, clip_scales, w0t, w1t, w2t, w3t):
    n = feats_T.shape[1]
    nb = 2048
    return pl.pallas_call(
        _mlp_body,
        grid=(n // nb,),
        in_specs=[
            pl.BlockSpec((_N_LEVELS, nb), lambda i: (0, i)),
            pl.BlockSpec((nb, 1), lambda i: (i, 0)),
            pl.BlockSpec((13, 256), lambda i: (0, 0)),
            pl.BlockSpec((256, 256), lambda i: (0, 0)),
            pl.BlockSpec((256, 256), lambda i: (0, 0)),
            pl.BlockSpec((256, 512), lambda i: (0, 0)),
        ],
        out_specs=[
            pl.BlockSpec((nb, 512), lambda i: (i, 0)),
            pl.BlockSpec((nb, _N_LEVELS), lambda i: (i, 0)),
        ],
        out_shape=[
            jax.ShapeDtypeStruct((n, 512), jnp.float32),
            jax.ShapeDtypeStruct((n, _N_LEVELS), jnp.float32),
        ],
    )(feats_T, clip_scales, w0t, w1t, w2t, w3t)


def kernel(positions, clip_scales, hash_table, W0, W1, W2, W3):
    pos_T = positions.T
    table_flat = hash_table.reshape(-1)
    u_T = _prep(pos_T)
    feats_T = _sc_encode(u_T, table_flat)
    clip, hashgrid = _mlp(feats_T, clip_scales.reshape(-1, 1),
                          W0.T, W1.T, W2.T, W3.T)
    return hashgrid, clip


# speedup vs baseline: 2.9405x; 1.0131x over previous
import numpy as np
import jax
import jax.numpy as jnp
from jax import lax
from jax.experimental import pallas as pl
from jax.experimental.pallas import tpu as pltpu
from jax.experimental.pallas import tpu_sc as plsc

_N_LEVELS = 12
_TSIZE = 2 ** 19
_BASE, _END = 16, 128
_GROWTH = float(np.exp((np.log(_END) - np.log(_BASE)) / (_N_LEVELS - 1)))
_RES = [int(np.floor(_BASE * _GROWTH ** l)) for l in range(_N_LEVELS)]
_P1 = int(np.uint32(2654435761).view(np.int32))
_P2 = int(np.uint32(805459861).view(np.int32))
_MASK = _TSIZE - 1

_NC, _NS = 2, 16
_NW = _NC * _NS
_C = 128
_ROWS = _N_LEVELS * 8
_FIRE = 8


_SPL = 0
_SROWS = _SPL * 8


_BC = 1024


def _sc_encode_body(nsub, pos_ref, tab_ref, out_ref,
                    posv, idxA, valA, idxB, valB, featv, stab, bounce,
                    semA, semB):
    cid = lax.axis_index("c")
    sid = lax.axis_index("s")
    wid = sid * _NC + cid
    pts_per_w = nsub * _C
    base0 = wid * pts_per_w
    if _SPL:
        sl = _SPL * _TSIZE // _NS

        def stage(ci, c0):
            off = sid * sl + ci * _BC
            pltpu.sync_copy(tab_ref.at[pl.ds(off, _BC)], bounce)
            pltpu.sync_copy(bounce, stab.at[pl.ds(off, _BC)])
            return c0

        lax.fori_loop(0, sl // _BC, stage, 0)
        plsc.subcore_barrier()
    pltpu.sync_copy(pos_ref.at[:, pl.ds(base0, pts_per_w)], posv)

    def compute(sub, idxv):
        def group_body(g, carry2):
            off = sub * _C + g * 16
            soff = g * 16
            ux = posv[0, pl.ds(off, 16)]
            uy = posv[1, pl.ds(off, 16)]
            uz = posv[2, pl.ds(off, 16)]
            for l in range(_N_LEVELS):
                r = jnp.float32(_RES[l])
                xi = (ux * r).astype(jnp.int32)
                yi = (uy * r).astype(jnp.int32)
                zi = (uz * r).astype(jnp.int32)
                hy0 = yi * jnp.int32(_P1)
                hy1 = hy0 + jnp.int32(_P1)
                hz0 = zi * jnp.int32(_P2)
                hz1 = hz0 + jnp.int32(_P2)
                xy = (xi ^ hy0, (xi + 1) ^ hy0, xi ^ hy1, (xi + 1) ^ hy1)
                lbase = jnp.int32(l * _TSIZE)
                c = 0
                for dz in (0, 1):
                    hz = hz0 if dz == 0 else hz1
                    for dy in (0, 1):
                        for dx in (0, 1):
                            h = ((xy[dy * 2 + dx] ^ hz) & jnp.int32(_MASK)) + lbase
                            idxv[l * 8 + c, pl.ds(soff, 16)] = h
                            c += 1
            return carry2

        lax.fori_loop(0, _C // 16, group_body, 0)

    def fire(idxv, valv, sem):
        if _SPL:
            def fs(j, c3):
                pltpu.make_async_copy(stab.at[idxv.at[j]], valv.at[j], sem).start()
                return c3

            lax.fori_loop(0, _SROWS, fs, 0)

        def f(j, c3):
            pltpu.make_async_copy(tab_ref.at[idxv.at[j]], valv.at[j], sem).start()
            return c3

        lax.fori_loop(_SROWS, _ROWS, f, 0)

    def drain_acc(sub, idxv, valv, sem):
        if _SPL:
            def drs(j, c3):
                pltpu.make_async_copy(stab.at[idxv.at[j]], valv.at[j], sem).wait()
                return c3

            lax.fori_loop(0, _SROWS, drs, 0)

        def dr(j, c3):
            pltpu.make_async_copy(tab_ref.at[idxv.at[j]], valv.at[j], sem).wait()
            return c3

        lax.fori_loop(_SROWS, _ROWS, dr, 0)

        def acc_body(g, carry2):
            off = g * 16
            ux = posv[0, pl.ds(sub * _C + off, 16)]
            uy = posv[1, pl.ds(sub * _C + off, 16)]
            uz = posv[2, pl.ds(sub * _C + off, 16)]
            for l in range(_N_LEVELS):
                r = jnp.float32(_RES[l])
                x = ux * r
                y = uy * r
                z = uz * r
                fx = x - x.astype(jnp.int32).astype(jnp.float32)
                fy = y - y.astype(jnp.int32).astype(jnp.float32)
                fz = z - z.astype(jnp.int32).astype(jnp.float32)
                gx = (1.0 - fx, fx)
                gy = (1.0 - fy, fy)
                gz = (1.0 - fz, fz)
                gxy = (gx[0] * gy[0], gx[1] * gy[0], gx[0] * gy[1], gx[1] * gy[1])
                acc = None
                c = 0
                for dz in (0, 1):
                    for dy in (0, 1):
                        for dx in (0, 1):
                            t = valv[l * 8 + c, pl.ds(off, 16)] * (gxy[dy * 2 + dx] * gz[dz])
                            acc = t if acc is None else acc + t
                            c += 1
                featv[l, pl.ds(off, 16)] = acc
            return carry2

        lax.fori_loop(0, _C // 16, acc_body, 0)
        pltpu.sync_copy(featv, out_ref.at[:, pl.ds(base0 + sub * _C, _C)])

    compute(0, idxA)
    fire(idxA, valA, semA)

    def pair(p, carry):
        s0 = 2 * p
        compute(s0 + 1, idxB)
        fire(idxB, valB, semB)
        drain_acc(s0, idxA, valA, semA)
        compute(s0 + 2, idxA)
        fire(idxA, valA, semA)
        drain_acc(s0 + 1, idxB, valB, semB)
        return carry

    lax.fori_loop(0, nsub // 2 - 1, pair, 0)
    compute(nsub - 1, idxB)
    fire(idxB, valB, semB)
    drain_acc(nsub - 2, idxA, valA, semA)
    drain_acc(nsub - 1, idxB, valB, semB)


def _sc_encode(pos_T, table_flat):
    n = pos_T.shape[1]
    nsub = n // (_NW * _C)
    mesh = plsc.VectorSubcoreMesh(core_axis_name="c", subcore_axis_name="s",
                                  num_cores=_NC, num_subcores=_NS)
    body = lambda *refs: _sc_encode_body(nsub, *refs)
    return pl.kernel(
        body,
        out_type=jax.ShapeDtypeStruct((_N_LEVELS, n), jnp.float32),
        mesh=mesh,
        scratch_types=[
            pltpu.VMEM((3, nsub * _C), jnp.float32),
            pltpu.VMEM((_ROWS, _C), jnp.int32),
            pltpu.VMEM((_ROWS, _C), jnp.float32),
            pltpu.VMEM((_ROWS, _C), jnp.int32),
            pltpu.VMEM((_ROWS, _C), jnp.float32),
            pltpu.VMEM((_N_LEVELS, _C), jnp.float32),
            pltpu.VMEM_SHARED((max(_SPL * _TSIZE, 16),), jnp.float32),
            pltpu.VMEM((_BC,), jnp.float32),
            pltpu.SemaphoreType.DMA,
            pltpu.SemaphoreType.DMA,
        ],
    )(pos_T, table_flat)


def _prep_body(pos_ref, out_ref):
    p = pos_ref[...]
    s = jnp.sum(p * p, axis=0, keepdims=True)
    rs = lax.rsqrt(jnp.maximum(s, 1.0))
    scale = jnp.where(s > 1.0, (2.0 - rs) * rs, 1.0)
    out_ref[...] = (p * scale + 2.0) * 0.25


def _prep(pos_T):
    n = pos_T.shape[1]
    nb = 8192
    return pl.pallas_call(
        _prep_body,
        grid=(n // nb,),
        in_specs=[pl.BlockSpec((3, nb), lambda i: (0, i))],
        out_specs=pl.BlockSpec((3, nb), lambda i: (0, i)),
        out_shape=jax.ShapeDtypeStruct((3, n), jnp.float32),
    )(pos_T)


def _mlp_body(feat_ref, cs_ref, w0_ref, w1_ref, w2_ref, w3_ref, clip_ref, hg_ref):
    ft = feat_ref[...]
    cs = cs_ref[...]
    w0 = w0_ref[...]
    h = lax.dot_general(ft, w0[0:12, :], (((0,), (0,)), ((), ())),
                        preferred_element_type=jnp.float32)
    h = jnp.maximum(h + cs * w0[12:13, :], 0.0)
    h = jnp.maximum(jnp.dot(h, w1_ref[...], preferred_element_type=jnp.float32), 0.0)
    h = jnp.maximum(jnp.dot(h, w2_ref[...], preferred_element_type=jnp.float32), 0.0)
    o = jnp.dot(h, w3_ref[...], preferred_element_type=jnp.float32)
    ssq = jnp.sum(o * o, axis=1, keepdims=True)
    clip_ref[...] = o * lax.rsqrt(ssq)
    hg_ref[...] = ft.T


def _mlp(feats_T, clip_scales, w0t, w1t, w2t, w3t):
    n = feats_T.shape[1]
    nb = 2048
    return pl.pallas_call(
        _mlp_body,
        grid=(n // nb,),
        in_specs=[
            pl.BlockSpec((_N_LEVELS, nb), lambda i: (0, i)),
            pl.BlockSpec((nb, 1), lambda i: (i, 0)),
            pl.BlockSpec((13, 256), lambda i: (0, 0)),
            pl.BlockSpec((256, 256), lambda i: (0, 0)),
            pl.BlockSpec((256, 256), lambda i: (0, 0)),
            pl.BlockSpec((256, 512), lambda i: (0, 0)),
        ],
        out_specs=[
            pl.BlockSpec((nb, 512), lambda i: (i, 0)),
            pl.BlockSpec((nb, _N_LEVELS), lambda i: (i, 0)),
        ],
        out_shape=[
            jax.ShapeDtypeStruct((n, 512), jnp.float32),
            jax.ShapeDtypeStruct((n, _N_LEVELS), jnp.float32),
        ],
    )(feats_T, clip_scales, w0t, w1t, w2t, w3t)


def kernel(positions, clip_scales, hash_table, W0, W1, W2, W3):
    pos_T = positions.T
    table_flat = hash_table.reshape(-1)
    u_T = _prep(pos_T)
    feats_T = _sc_encode(u_T, table_flat)
    clip, hashgrid = _mlp(feats_T, clip_scales.reshape(-1, 1),
                          W0.T, W1.T, W2.T, W3.T)
    return hashgrid, clip


# speedup vs baseline: 3.7794x; 1.2853x over previous
import numpy as np
import jax
import jax.numpy as jnp
from jax import lax
from jax.experimental import pallas as pl
from jax.experimental.pallas import tpu as pltpu
from jax.experimental.pallas import tpu_sc as plsc

_N_LEVELS = 12
_TSIZE = 2 ** 19
_BASE, _END = 16, 128
_GROWTH = float(np.exp((np.log(_END) - np.log(_BASE)) / (_N_LEVELS - 1)))
_RES = [int(np.floor(_BASE * _GROWTH ** l)) for l in range(_N_LEVELS)]
_P1 = int(np.uint32(2654435761).view(np.int32))
_P2 = int(np.uint32(805459861).view(np.int32))
_MASK = _TSIZE - 1

_NC, _NS = 2, 16
_NW = _NC * _NS
_C = 128

_NDL = 3
_ZP = 32
_DMETA = []
_doff = 0
for _l in range(_NDL):
    _D = _RES[_l] + 2
    _ng = _D * _D * (_ZP // 16)
    _nr = -(-_ng // 8)
    _DMETA.append((_RES[_l], _D, _doff, _ng, _nr))
    _doff += _nr * 128
_DTOT = _doff
_SL = _N_LEVELS - _NDL
_SRW = _SL * 8


def _sc_encode_body(nsub, pos_ref, tab_ref, out_ref,
                    posv, idxA, valA, idxB, valB, featv, dense, stab,
                    semA, semB):
    cid = lax.axis_index("c")
    sid = lax.axis_index("s")
    wid = sid * _NC + cid
    pts_per_w = nsub * _C
    base0 = wid * pts_per_w

    for l in range(_NDL):
        res, d, off, ngroups, nrows = _DMETA[l]
        inv_d = np.float32(1.0 / d)

        def row_body(k, c0, res=res, d=d, off=off, ngroups=ngroups,
                     nrows=nrows, inv_d=inv_d, l=l):
            rc = jnp.minimum(k * _NS + sid, nrows - 1)
            for i in range(8):
                gi = jnp.minimum(rc * 8 + i, ngroups - 1)
                giv = jnp.full((16,), gi, jnp.int32)
                zg = giv & jnp.int32(1)
                xy = lax.shift_right_logical(giv, 1)
                xyf = xy.astype(jnp.float32) + 0.5
                x = (xyf * inv_d).astype(jnp.int32)
                y = xy - x * jnp.int32(d)
                z = zg * 16 + lax.iota(jnp.int32, 16)
                zc = jnp.minimum(z, jnp.int32(res + 1))
                h = ((x ^ (y * jnp.int32(_P1)) ^ (zc * jnp.int32(_P2)))
                     & jnp.int32(_MASK)) + jnp.int32(l * _TSIZE)
                idxA[0, pl.ds(i * 16, 16)] = h
            pltpu.make_async_copy(tab_ref.at[idxA.at[0]], valA.at[0],
                                  semA).start()
            pltpu.make_async_copy(tab_ref.at[idxA.at[0]], valA.at[0],
                                  semA).wait()
            pltpu.sync_copy(valA.at[0], stab.at[off // 128 + rc])
            return c0

        lax.fori_loop(0, -(-nrows // _NS), row_body, 0)
    plsc.subcore_barrier()
    pltpu.sync_copy(stab, dense)

    pltpu.sync_copy(pos_ref.at[:, pl.ds(base0, pts_per_w)], posv)

    def compute(sub, idxv):
        def group_body(g, carry2):
            off = sub * _C + g * 16
            soff = g * 16
            ux = posv[0, pl.ds(off, 16)]
            uy = posv[1, pl.ds(off, 16)]
            uz = posv[2, pl.ds(off, 16)]
            for l in range(_NDL, _N_LEVELS):
                r = jnp.float32(_RES[l])
                xi = (ux * r).astype(jnp.int32)
                yi = (uy * r).astype(jnp.int32)
                zi = (uz * r).astype(jnp.int32)
                hy0 = yi * jnp.int32(_P1)
                hy1 = hy0 + jnp.int32(_P1)
                hz0 = zi * jnp.int32(_P2)
                hz1 = hz0 + jnp.int32(_P2)
                xy = (xi ^ hy0, (xi + 1) ^ hy0, xi ^ hy1, (xi + 1) ^ hy1)
                lbase = jnp.int32(l * _TSIZE)
                c = 0
                for dz in (0, 1):
                    hz = hz0 if dz == 0 else hz1
                    for dy in (0, 1):
                        for dx in (0, 1):
                            h = ((xy[dy * 2 + dx] ^ hz) & jnp.int32(_MASK)) + lbase
                            idxv[(l - _NDL) * 8 + c, pl.ds(soff, 16)] = h
                            c += 1
            return carry2

        lax.fori_loop(0, _C // 16, group_body, 0)

    def fire(idxv, valv, sem):
        def f(j, c3):
            pltpu.make_async_copy(tab_ref.at[idxv.at[j]], valv.at[j], sem).start()
            return c3

        lax.fori_loop(0, _SRW, f, 0)

    def drain_acc(sub, idxv, valv, sem):
        def dr(j, c3):
            pltpu.make_async_copy(tab_ref.at[idxv.at[j]], valv.at[j], sem).wait()
            return c3

        lax.fori_loop(0, _SRW, dr, 0)

        def acc_body(g, carry2):
            off = g * 16
            ux = posv[0, pl.ds(sub * _C + off, 16)]
            uy = posv[1, pl.ds(sub * _C + off, 16)]
            uz = posv[2, pl.ds(sub * _C + off, 16)]
            for l in range(_N_LEVELS):
                r = jnp.float32(_RES[l])
                x = ux * r
                y = uy * r
                z = uz * r
                xi = x.astype(jnp.int32)
                yi = y.astype(jnp.int32)
                zi = z.astype(jnp.int32)
                fx = x - xi.astype(jnp.float32)
                fy = y - yi.astype(jnp.float32)
                fz = z - zi.astype(jnp.float32)
                gx = (1.0 - fx, fx)
                gy = (1.0 - fy, fy)
                gz = (1.0 - fz, fz)
                gxy = (gx[0] * gy[0], gx[1] * gy[0], gx[0] * gy[1], gx[1] * gy[1])
                if l < _NDL:
                    res, d, doff, _, _ = _DMETA[l]
                    i0 = ((xi * jnp.int32(d) + yi) * jnp.int32(_ZP) + zi
                          + jnp.int32(doff))
                else:
                    i0 = None
                acc = None
                c = 0
                for dz in (0, 1):
                    for dy in (0, 1):
                        for dx in (0, 1):
                            if i0 is not None:
                                ic = i0 + jnp.int32(dx * d * _ZP + dy * _ZP + dz)
                                v = plsc.load_gather(
                                    dense,
                                    [lax.shift_right_logical(ic, 7),
                                     ic & jnp.int32(127)])
                            else:
                                v = valv[(l - _NDL) * 8 + c, pl.ds(off, 16)]
                            t = v * (gxy[dy * 2 + dx] * gz[dz])
                            acc = t if acc is None else acc + t
                            c += 1
                featv[l, pl.ds(off, 16)] = acc
            return carry2

        lax.fori_loop(0, _C // 16, acc_body, 0)
        pltpu.sync_copy(featv, out_ref.at[:, pl.ds(base0 + sub * _C, _C)])

    compute(0, idxA)
    fire(idxA, valA, semA)

    def pair(p, carry):
        s0 = 2 * p
        compute(s0 + 1, idxB)
        fire(idxB, valB, semB)
        drain_acc(s0, idxA, valA, semA)
        compute(s0 + 2, idxA)
        fire(idxA, valA, semA)
        drain_acc(s0 + 1, idxB, valB, semB)
        return carry

    lax.fori_loop(0, nsub // 2 - 1, pair, 0)
    compute(nsub - 1, idxB)
    fire(idxB, valB, semB)
    drain_acc(nsub - 2, idxA, valA, semA)
    drain_acc(nsub - 1, idxB, valB, semB)


def _sc_encode(pos_T, table_flat):
    n = pos_T.shape[1]
    nsub = n // (_NW * _C)
    mesh = plsc.VectorSubcoreMesh(core_axis_name="c", subcore_axis_name="s",
                                  num_cores=_NC, num_subcores=_NS)
    body = lambda *refs: _sc_encode_body(nsub, *refs)
    return pl.kernel(
        body,
        out_type=jax.ShapeDtypeStruct((_N_LEVELS, n), jnp.float32),
        mesh=mesh,
        compiler_params=pltpu.CompilerParams(needs_layout_passes=False),
        scratch_types=[
            pltpu.VMEM((3, nsub * _C), jnp.float32),
            pltpu.VMEM((_SRW, _C), jnp.int32),
            pltpu.VMEM((_SRW, _C), jnp.float32),
            pltpu.VMEM((_SRW, _C), jnp.int32),
            pltpu.VMEM((_SRW, _C), jnp.float32),
            pltpu.VMEM((_N_LEVELS, _C), jnp.float32),
            pltpu.VMEM((_DTOT // 128, 128), jnp.float32),
            pltpu.VMEM_SHARED((_DTOT // 128, 128), jnp.float32),
            pltpu.SemaphoreType.DMA,
            pltpu.SemaphoreType.DMA,
        ],
    )(pos_T, table_flat)


def _prep_body(pos_ref, out_ref):
    p = pos_ref[...]
    s = jnp.sum(p * p, axis=0, keepdims=True)
    rs = lax.rsqrt(jnp.maximum(s, 1.0))
    scale = jnp.where(s > 1.0, (2.0 - rs) * rs, 1.0)
    out_ref[...] = (p * scale + 2.0) * 0.25


def _prep(pos_T):
    n = pos_T.shape[1]
    nb = 8192
    return pl.pallas_call(
        _prep_body,
        grid=(n // nb,),
        in_specs=[pl.BlockSpec((3, nb), lambda i: (0, i))],
        out_specs=pl.BlockSpec((3, nb), lambda i: (0, i)),
        out_shape=jax.ShapeDtypeStruct((3, n), jnp.float32),
    )(pos_T)


def _mlp_body(feat_ref, cs_ref, w0_ref, w1_ref, w2_ref, w3_ref, clip_ref, hg_ref):
    ft = feat_ref[...]
    cs = cs_ref[...]
    w0 = w0_ref[...]
    h = lax.dot_general(ft, w0[0:12, :], (((0,), (0,)), ((), ())),
                        preferred_element_type=jnp.float32)
    h = jnp.maximum(h + cs * w0[12:13, :], 0.0)
    h = jnp.maximum(jnp.dot(h, w1_ref[...], preferred_element_type=jnp.float32), 0.0)
    h = jnp.maximum(jnp.dot(h, w2_ref[...], preferred_element_type=jnp.float32), 0.0)
    o = jnp.dot(h, w3_ref[...], preferred_element_type=jnp.float32)
    ssq = jnp.sum(o * o, axis=1, keepdims=True)
    clip_ref[...] = o * lax.rsqrt(ssq)
    hg_ref[...] = ft.T


def _mlp(feats_T, clip_scales, w0t, w1t, w2t, w3t):
    n = feats_T.shape[1]
    nb = 2048
    return pl.pallas_call(
        _mlp_body,
        grid=(n // nb,),
        in_specs=[
            pl.BlockSpec((_N_LEVELS, nb), lambda i: (0, i)),
            pl.BlockSpec((nb, 1), lambda i: (i, 0)),
            pl.BlockSpec((13, 256), lambda i: (0, 0)),
            pl.BlockSpec((256, 256), lambda i: (0, 0)),
            pl.BlockSpec((256, 256), lambda i: (0, 0)),
            pl.BlockSpec((256, 512), lambda i: (0, 0)),
        ],
        out_specs=[
            pl.BlockSpec((nb, 512), lambda i: (i, 0)),
            pl.BlockSpec((nb, _N_LEVELS), lambda i: (i, 0)),
        ],
        out_shape=[
            jax.ShapeDtypeStruct((n, 512), jnp.float32),
            jax.ShapeDtypeStruct((n, _N_LEVELS), jnp.float32),
        ],
    )(feats_T, clip_scales, w0t, w1t, w2t, w3t)


def kernel(positions, clip_scales, hash_table, W0, W1, W2, W3):
    pos_T = positions.T
    table_flat = hash_table.reshape(-1)
    u_T = _prep(pos_T)
    feats_T = _sc_encode(u_T, table_flat)
    clip, hashgrid = _mlp(feats_T, clip_scales.reshape(-1, 1),
                          W0.T, W1.T, W2.T, W3.T)
    return hashgrid, clip


# speedup vs baseline: 3.7837x; 1.0012x over previous
import numpy as np
import jax
import jax.numpy as jnp
from jax import lax
from jax.experimental import pallas as pl
from jax.experimental.pallas import tpu as pltpu
from jax.experimental.pallas import tpu_sc as plsc

_N_LEVELS = 12
_TSIZE = 2 ** 19
_BASE, _END = 16, 128
_GROWTH = float(np.exp((np.log(_END) - np.log(_BASE)) / (_N_LEVELS - 1)))
_RES = [int(np.floor(_BASE * _GROWTH ** l)) for l in range(_N_LEVELS)]
_P1 = int(np.uint32(2654435761).view(np.int32))
_P2 = int(np.uint32(805459861).view(np.int32))
_MASK = _TSIZE - 1

_NC, _NS = 2, 16
_NW = _NC * _NS
_C = 128

_NDL = 3
_ZP = 32
_DMETA = []
_doff = 0
for _l in range(_NDL):
    _D = _RES[_l] + 2
    _ng = _D * _D * (_ZP // 16)
    _nr = -(-_ng // 8)
    _DMETA.append((_RES[_l], _D, _doff, _ng, _nr))
    _doff += _nr * 128
_DTOT = _doff
_SL = _N_LEVELS - _NDL
_SRW = _SL * 8


def _sc_encode_body(nsub, pos_ref, tab_ref, out_ref,
                    posv, idxA, valA, idxB, valB, featv, dense, stab,
                    semA, semB):
    cid = lax.axis_index("c")
    sid = lax.axis_index("s")
    wid = sid * _NC + cid
    pts_per_w = nsub * _C
    base0 = wid * pts_per_w

    for l in range(_NDL):
        res, d, off, ngroups, nrows = _DMETA[l]
        inv_d = np.float32(1.0 / d)

        def row_body(k, c0, res=res, d=d, off=off, ngroups=ngroups,
                     nrows=nrows, inv_d=inv_d, l=l):
            rc = jnp.minimum(k * _NS + sid, nrows - 1)
            for i in range(8):
                gi = jnp.minimum(rc * 8 + i, ngroups - 1)
                giv = jnp.full((16,), gi, jnp.int32)
                zg = giv & jnp.int32(1)
                xy = lax.shift_right_logical(giv, 1)
                xyf = xy.astype(jnp.float32) + 0.5
                x = (xyf * inv_d).astype(jnp.int32)
                y = xy - x * jnp.int32(d)
                z = zg * 16 + lax.iota(jnp.int32, 16)
                zc = jnp.minimum(z, jnp.int32(res + 1))
                h = ((x ^ (y * jnp.int32(_P1)) ^ (zc * jnp.int32(_P2)))
                     & jnp.int32(_MASK)) + jnp.int32(l * _TSIZE)
                idxA[0, pl.ds(i * 16, 16)] = h
            pltpu.make_async_copy(tab_ref.at[idxA.at[0]], valA.at[0],
                                  semA).start()
            pltpu.make_async_copy(tab_ref.at[idxA.at[0]], valA.at[0],
                                  semA).wait()
            pltpu.sync_copy(valA.at[0], stab.at[off // 128 + rc])
            return c0

        lax.fori_loop(0, -(-nrows // _NS), row_body, 0)
    plsc.subcore_barrier()
    pltpu.sync_copy(stab, dense)

    pltpu.sync_copy(pos_ref.at[:, pl.ds(base0, pts_per_w)], posv)

    def compute(sub, idxv):
        def group_body(g, carry2):
            off = sub * _C + g * 16
            soff = g * 16
            ux = posv[0, pl.ds(off, 16)]
            uy = posv[1, pl.ds(off, 16)]
            uz = posv[2, pl.ds(off, 16)]
            for l in range(_NDL, _N_LEVELS):
                r = jnp.float32(_RES[l])
                xi = (ux * r).astype(jnp.int32)
                yi = (uy * r).astype(jnp.int32)
                zi = (uz * r).astype(jnp.int32)
                hy0 = yi * jnp.int32(_P1)
                hy1 = hy0 + jnp.int32(_P1)
                hz0 = zi * jnp.int32(_P2)
                hz1 = hz0 + jnp.int32(_P2)
                xy = (xi ^ hy0, (xi + 1) ^ hy0, xi ^ hy1, (xi + 1) ^ hy1)
                lbase = jnp.int32(l * _TSIZE)
                c = 0
                for dz in (0, 1):
                    hz = hz0 if dz == 0 else hz1
                    for dy in (0, 1):
                        for dx in (0, 1):
                            h = ((xy[dy * 2 + dx] ^ hz) & jnp.int32(_MASK)) + lbase
                            idxv[(l - _NDL) * 8 + c, pl.ds(soff, 16)] = h
                            c += 1
            return carry2

        lax.fori_loop(0, _C // 16, group_body, 0)

    def fire(idxv, valv, sem):
        def f(j, c3):
            pltpu.make_async_copy(tab_ref.at[idxv.at[j]], valv.at[j], sem).start()
            return c3

        lax.fori_loop(0, _SRW, f, 0)

    def drain_acc(sub, idxv, valv, sem):
        def dr(j, c3):
            pltpu.make_async_copy(tab_ref.at[idxv.at[j]], valv.at[j], sem).wait()
            return c3

        lax.fori_loop(0, _SRW, dr, 0)

        def acc_body(g, carry2):
            off = g * 16
            ux = posv[0, pl.ds(sub * _C + off, 16)]
            uy = posv[1, pl.ds(sub * _C + off, 16)]
            uz = posv[2, pl.ds(sub * _C + off, 16)]
            for l in range(_N_LEVELS):
                r = jnp.float32(_RES[l])
                x = ux * r
                y = uy * r
                z = uz * r
                xi = x.astype(jnp.int32)
                yi = y.astype(jnp.int32)
                zi = z.astype(jnp.int32)
                fx = x - xi.astype(jnp.float32)
                fy = y - yi.astype(jnp.float32)
                fz = z - zi.astype(jnp.float32)
                gx = (1.0 - fx, fx)
                gy = (1.0 - fy, fy)
                gz = (1.0 - fz, fz)
                gxy = (gx[0] * gy[0], gx[1] * gy[0], gx[0] * gy[1], gx[1] * gy[1])
                if l < _NDL:
                    res, d, doff, _, _ = _DMETA[l]
                    i0 = ((xi * jnp.int32(d) + yi) * jnp.int32(_ZP) + zi
                          + jnp.int32(doff))
                else:
                    i0 = None
                acc = None
                c = 0
                for dz in (0, 1):
                    for dy in (0, 1):
                        for dx in (0, 1):
                            if i0 is not None:
                                ic = i0 + jnp.int32(dx * d * _ZP + dy * _ZP + dz)
                                v = plsc.load_gather(
                                    dense,
                                    [lax.shift_right_logical(ic, 7),
                                     ic & jnp.int32(127)])
                            else:
                                v = valv[(l - _NDL) * 8 + c, pl.ds(off, 16)]
                            t = v * (gxy[dy * 2 + dx] * gz[dz])
                            acc = t if acc is None else acc + t
                            c += 1
                featv[l, pl.ds(off, 16)] = acc
            return carry2

        lax.fori_loop(0, _C // 16, acc_body, 0)
        pltpu.sync_copy(featv, out_ref.at[:, pl.ds(base0 + sub * _C, _C)])

    compute(0, idxA)
    fire(idxA, valA, semA)

    def pair(p, carry):
        s0 = 2 * p
        compute(s0 + 1, idxB)
        fire(idxB, valB, semB)
        drain_acc(s0, idxA, valA, semA)
        compute(s0 + 2, idxA)
        fire(idxA, valA, semA)
        drain_acc(s0 + 1, idxB, valB, semB)
        return carry

    lax.fori_loop(0, nsub // 2 - 1, pair, 0)
    compute(nsub - 1, idxB)
    fire(idxB, valB, semB)
    drain_acc(nsub - 2, idxA, valA, semA)
    drain_acc(nsub - 1, idxB, valB, semB)


def _sc_encode(pos_T, table_flat):
    n = pos_T.shape[1]
    nsub = n // (_NW * _C)
    mesh = plsc.VectorSubcoreMesh(core_axis_name="c", subcore_axis_name="s",
                                  num_cores=_NC, num_subcores=_NS)
    body = lambda *refs: _sc_encode_body(nsub, *refs)
    return pl.kernel(
        body,
        out_type=jax.ShapeDtypeStruct((_N_LEVELS, n), jnp.float32),
        mesh=mesh,
        compiler_params=pltpu.CompilerParams(needs_layout_passes=False),
        scratch_types=[
            pltpu.VMEM((3, nsub * _C), jnp.float32),
            pltpu.VMEM((_SRW, _C), jnp.int32),
            pltpu.VMEM((_SRW, _C), jnp.float32),
            pltpu.VMEM((_SRW, _C), jnp.int32),
            pltpu.VMEM((_SRW, _C), jnp.float32),
            pltpu.VMEM((_N_LEVELS, _C), jnp.float32),
            pltpu.VMEM((_DTOT // 128, 128), jnp.float32),
            pltpu.VMEM_SHARED((_DTOT // 128, 128), jnp.float32),
            pltpu.SemaphoreType.DMA,
            pltpu.SemaphoreType.DMA,
        ],
    )(pos_T, table_flat)


def _prep_body(pos_ref, out_ref):
    p = pos_ref[...]
    s = jnp.sum(p * p, axis=0, keepdims=True)
    rs = lax.rsqrt(jnp.maximum(s, 1.0))
    scale = jnp.where(s > 1.0, (2.0 - rs) * rs, 1.0)
    out_ref[...] = (p * scale + 2.0) * 0.25


def _prep(pos_T):
    n = pos_T.shape[1]
    nb = 8192
    return pl.pallas_call(
        _prep_body,
        grid=(n // nb,),
        in_specs=[pl.BlockSpec((3, nb), lambda i: (0, i))],
        out_specs=pl.BlockSpec((3, nb), lambda i: (0, i)),
        out_shape=jax.ShapeDtypeStruct((3, n), jnp.float32),
    )(pos_T)


def _mlp_body(feat_ref, cs_ref, w0_ref, w1_ref, w2_ref, w3_ref, clip_ref, hg_ref):
    ft = feat_ref[...]
    cs = cs_ref[...]
    w0 = w0_ref[...]
    prec = lax.Precision.DEFAULT
    h = lax.dot_general(ft, w0[0:12, :], (((0,), (0,)), ((), ())),
                        precision=prec, preferred_element_type=jnp.float32)
    h = jnp.maximum(h + cs * w0[12:13, :], 0.0)
    h = jnp.maximum(jnp.dot(h, w1_ref[...], precision=prec,
                            preferred_element_type=jnp.float32), 0.0)
    h = jnp.maximum(jnp.dot(h, w2_ref[...], precision=prec,
                            preferred_element_type=jnp.float32), 0.0)
    o = jnp.dot(h, w3_ref[...], precision=prec,
                preferred_element_type=jnp.float32)
    ssq = jnp.sum(o * o, axis=1, keepdims=True)
    clip_ref[...] = o * lax.rsqrt(ssq)
    hg_ref[...] = ft.T


def _mlp(feats_T, clip_scales, w0t, w1t, w2t, w3t):
    n = feats_T.shape[1]
    nb = 2048
    return pl.pallas_call(
        _mlp_body,
        grid=(n // nb,),
        in_specs=[
            pl.BlockSpec((_N_LEVELS, nb), lambda i: (0, i)),
            pl.BlockSpec((nb, 1), lambda i: (i, 0)),
            pl.BlockSpec((13, 256), lambda i: (0, 0)),
            pl.BlockSpec((256, 256), lambda i: (0, 0)),
            pl.BlockSpec((256, 256), lambda i: (0, 0)),
            pl.BlockSpec((256, 512), lambda i: (0, 0)),
        ],
        out_specs=[
            pl.BlockSpec((nb, 512), lambda i: (i, 0)),
            pl.BlockSpec((nb, _N_LEVELS), lambda i: (i, 0)),
        ],
        out_shape=[
            jax.ShapeDtypeStruct((n, 512), jnp.float32),
            jax.ShapeDtypeStruct((n, _N_LEVELS), jnp.float32),
        ],
    )(feats_T, clip_scales, w0t, w1t, w2t, w3t)


def kernel(positions, clip_scales, hash_table, W0, W1, W2, W3):
    pos_T = positions.T
    table_flat = hash_table.reshape(-1)
    u_T = _prep(pos_T)
    feats_T = _sc_encode(u_T, table_flat)
    clip, hashgrid = _mlp(feats_T, clip_scales.reshape(-1, 1),
                          W0.T, W1.T, W2.T, W3.T)
    return hashgrid, clip
